# baseline (speedup 1.0000x reference)
import functools

import jax
import jax.numpy as jnp
from jax.experimental import pallas as pl
from jax.experimental.pallas import tpu as pltpu

_NT = 2048


def _chamfer_kernel(nchunk, n, m,
                    a_ref, b_ref, out_ref,
                    bt_ref, n2_ref, d1_ref, s0_ref, acc_ref):
    b = pl.program_id(0)
    ni = pl.program_id(1)

    a = a_ref[0]

    @pl.when(ni == 0)
    def _per_batch_setup():
        bt = jnp.transpose(b_ref[0], (1, 0))
        bt_ref[...] = bt
        n2_ref[...] = jnp.sum(bt * bt, axis=0, keepdims=True)

    bt = bt_ref[...]
    n1 = jnp.sum(a * a, axis=1, keepdims=True)
    c2 = jax.lax.dot_general(
        (a * (-2.0)).astype(jnp.bfloat16), bt.astype(jnp.bfloat16),
        dimension_numbers=(((1,), (0,)), ((), ())),
        preferred_element_type=jnp.float32,
    )
    d2 = (n1 + n2_ref[...]) + c2

    rowmin = jnp.maximum(jnp.min(d2, axis=1, keepdims=True), 0.0)
    colmin = jnp.min(d2, axis=0, keepdims=True)
    sq = jnp.sqrt(rowmin)

    @pl.when(ni == 0)
    def _init():
        d1_ref[...] = colmin
        s0_ref[...] = sq

    @pl.when(ni > 0)
    def _accum():
        d1_ref[...] = jnp.minimum(d1_ref[...], colmin)
        s0_ref[...] = s0_ref[...] + sq

    @pl.when(ni == nchunk - 1)
    def _finish():
        s1 = jnp.sum(jnp.sqrt(jnp.maximum(d1_ref[...], 0.0)))
        lb = jnp.maximum(jnp.sum(s0_ref[...]) / n, s1 / m)

        @pl.when(b == 0)
        def _first():
            acc_ref[...] = jnp.full_like(acc_ref, 0.0) + lb

        @pl.when(b > 0)
        def _rest():
            acc_ref[...] = acc_ref[...] + lb

        out_ref[...] = acc_ref[...] / pl.num_programs(0)


def kernel(input1, input2):
    bsz, n, d = input1.shape
    m = input2.shape[1]
    nchunk = n // _NT

    body = functools.partial(_chamfer_kernel, nchunk, n, m)
    out = pl.pallas_call(
        body,
        grid=(bsz, nchunk),
        in_specs=[
            pl.BlockSpec((1, _NT, d), lambda b, ni: (b, ni, 0)),
            pl.BlockSpec((1, m, d), lambda b, ni: (b, 0, 0)),
        ],
        out_specs=pl.BlockSpec((1, 1), lambda b, ni: (0, 0)),
        out_shape=jax.ShapeDtypeStruct((1, 1), jnp.float32),
        scratch_shapes=[
            pltpu.VMEM((d, m), jnp.float32),
            pltpu.VMEM((1, m), jnp.float32),
            pltpu.VMEM((1, m), jnp.float32),
            pltpu.VMEM((_NT, 1), jnp.float32),
            pltpu.VMEM((1, 1), jnp.float32),
        ],
        compiler_params=pltpu.CompilerParams(
            dimension_semantics=("arbitrary", "arbitrary"),
        ),
    )(input1, input2)
    return out[0, 0]


# speedup vs baseline: 1.0013x; 1.0013x over previous
import functools

import jax
import jax.numpy as jnp
from jax.experimental import pallas as pl
from jax.experimental.pallas import tpu as pltpu

_NT = 2048


def _chamfer_kernel(nchunk, n, m,
                    a_ref, bt_ref, out_ref,
                    n2_ref, d1_ref, s0_ref, acc_ref):
    b = pl.program_id(0)
    ni = pl.program_id(1)

    a = a_ref[0]
    bt = bt_ref[0]

    @pl.when(ni == 0)
    def _per_batch_setup():
        n2_ref[...] = jnp.sum(bt * bt, axis=0, keepdims=True)

    n1 = jnp.sum(a * a, axis=1, keepdims=True)
    c2 = jax.lax.dot_general(
        (a * (-2.0)).astype(jnp.bfloat16), bt.astype(jnp.bfloat16),
        dimension_numbers=(((1,), (0,)), ((), ())),
        preferred_element_type=jnp.float32,
    )
    d2 = (n1 + n2_ref[...]) + c2

    rowmin = jnp.maximum(jnp.min(d2, axis=1, keepdims=True), 0.0)
    colmin = jnp.min(d2, axis=0, keepdims=True)
    sq = jnp.sqrt(rowmin)

    @pl.when(ni == 0)
    def _init():
        d1_ref[...] = colmin
        s0_ref[...] = sq

    @pl.when(ni > 0)
    def _accum():
        d1_ref[...] = jnp.minimum(d1_ref[...], colmin)
        s0_ref[...] = s0_ref[...] + sq

    @pl.when(ni == nchunk - 1)
    def _finish():
        s1 = jnp.sum(jnp.sqrt(jnp.maximum(d1_ref[...], 0.0)))
        lb = jnp.maximum(jnp.sum(s0_ref[...]) / n, s1 / m)

        @pl.when(b == 0)
        def _first():
            acc_ref[...] = jnp.full_like(acc_ref, 0.0) + lb

        @pl.when(b > 0)
        def _rest():
            acc_ref[...] = acc_ref[...] + lb

        out_ref[...] = acc_ref[...] / pl.num_programs(0)


def kernel(input1, input2):
    bsz, n, d = input1.shape
    m = input2.shape[1]
    nchunk = n // _NT
    i2t = jnp.transpose(input2, (0, 2, 1))

    body = functools.partial(_chamfer_kernel, nchunk, n, m)
    out = pl.pallas_call(
        body,
        grid=(bsz, nchunk),
        in_specs=[
            pl.BlockSpec((1, _NT, d), lambda b, ni: (b, ni, 0)),
            pl.BlockSpec((1, d, m), lambda b, ni: (b, 0, 0)),
        ],
        out_specs=pl.BlockSpec((1, 1), lambda b, ni: (0, 0)),
        out_shape=jax.ShapeDtypeStruct((1, 1), jnp.float32),
        scratch_shapes=[
            pltpu.VMEM((1, m), jnp.float32),
            pltpu.VMEM((1, m), jnp.float32),
            pltpu.VMEM((_NT, 1), jnp.float32),
            pltpu.VMEM((1, 1), jnp.float32),
        ],
        compiler_params=pltpu.CompilerParams(
            dimension_semantics=("arbitrary", "arbitrary"),
        ),
    )(input1, i2t)
    return out[0, 0]
